# baseline (speedup 1.0000x reference)
import functools

import numpy as np
import jax
import jax.numpy as jnp
from jax.experimental import pallas as pl
import jax.experimental.pallas.tpu as pltpu

N, F_IN, H, C = 10000, 128, 128, 64
K = 10
BM = 1024
NP = K * BM
EDGE = N - (K - 1) * BM

_COL_AR, _COL_AC, _COL_IB, _COL_L1, _COL_L1I, _COL_FIN, _COL_L2, _COL_L2I, \
    _COL_EMIT, _COL_OUTB = range(10)


def _build_schedule() -> np.ndarray:
    rows = []
    seen_l2 = set()

    def add(ar, ac, ib, l1, l1init, fin, l2, emit, outb):
        l2init = 1 if (l2 and ib not in seen_l2) else 0
        if l2:
            seen_l2.add(ib)
        rows.append([ar, ac, ib, l1, l1init, fin, l2, l2init, emit, outb])

    for i in range(K):
        for p in range(K):
            j = (i + 1 + p) % K
            fin = 1 if p == K - 1 else 0
            l2 = 1 if (j < i or fin) else 0
            add(i, j, i, 1, 1 if p == 0 else 0, fin, l2, 0, 0)
    for i in range(K - 1):
        for j in range(i + 1, K):
            add(i, j, i, 0, 0, 0, 1, 1 if j == K - 1 else 0, i)
    add(K - 2, K - 1, K - 1, 0, 0, 0, 0, 1, K - 1)
    return np.asarray(rows, dtype=np.int32)


_SCHEDULE = _build_schedule()
_STEPS = _SCHEDULE.shape[0]


def _gcn_kernel(sref, adj_ref, x_ref, w1_ref, w2_ref, b1_ref, b2_ref,
                out_ref, s1_ref, acc_ref, t_ref, oacc_ref):
    step = pl.program_id(0)
    ib = sref[step, _COL_IB]
    jb = sref[step, _COL_AC]

    @pl.when(step == 0)
    def _():
        s1_ref[pl.ds(0, N), :] = jnp.dot(
            x_ref[...], w1_ref[...],
            preferred_element_type=jnp.float32).astype(jnp.bfloat16)
        s1_ref[pl.ds(N, NP - N), :] = jnp.zeros((NP - N, H), jnp.bfloat16)

    col_limit = jnp.where(jb == K - 1, EDGE, BM)
    col_ok = jax.lax.broadcasted_iota(jnp.int32, (BM, BM), 1) < col_limit
    adj_bf = jnp.where(col_ok, adj_ref[...], 0.0).astype(jnp.bfloat16)

    @pl.when(sref[step, _COL_L1] == 1)
    def _():
        c = jnp.dot(adj_bf, s1_ref[pl.ds(jb * BM, BM), :],
                    preferred_element_type=jnp.float32)

        @pl.when(sref[step, _COL_L1I] == 1)
        def _():
            acc_ref[...] = c

        @pl.when(sref[step, _COL_L1I] == 0)
        def _():
            acc_ref[...] += c

    @pl.when(sref[step, _COL_FIN] == 1)
    def _():
        h1 = jnp.maximum(acc_ref[...] + b1_ref[...], 0.0)
        tv = jnp.dot(h1.astype(jnp.bfloat16), w2_ref[...].astype(jnp.bfloat16),
                     preferred_element_type=jnp.float32)
        row_limit = jnp.where(ib == K - 1, EDGE, BM)
        row_ok = jax.lax.broadcasted_iota(jnp.int32, (BM, C), 0) < row_limit
        t_ref[pl.ds(ib * BM, BM), :] = jnp.where(
            row_ok, tv, 0.0).astype(jnp.bfloat16)

    @pl.when(sref[step, _COL_L2] == 1)
    def _():
        c2 = jnp.dot(adj_bf, t_ref[pl.ds(jb * BM, BM), :],
                     preferred_element_type=jnp.float32)

        @pl.when(sref[step, _COL_L2I] == 1)
        def _():
            oacc_ref[pl.ds(ib * BM, BM), :] = c2

        @pl.when(sref[step, _COL_L2I] == 0)
        def _():
            oacc_ref[pl.ds(ib * BM, BM), :] += c2

    @pl.when(sref[step, _COL_EMIT] == 1)
    def _():
        z = oacc_ref[pl.ds(ib * BM, BM), :] + b2_ref[...]
        zs = z - jnp.max(z, axis=1, keepdims=True)
        out_ref[...] = zs - jnp.log(
            jnp.sum(jnp.exp(zs), axis=1, keepdims=True))


@jax.jit
def kernel(x, adj, W1, b1, W2, b2):
    grid_spec = pltpu.PrefetchScalarGridSpec(
        num_scalar_prefetch=1,
        grid=(_STEPS,),
        in_specs=[
            pl.BlockSpec((BM, BM),
                         lambda s, sref: (sref[s, _COL_AR], sref[s, _COL_AC])),
            pl.BlockSpec((N, F_IN), lambda s, sref: (0, 0)),
            pl.BlockSpec((F_IN, H), lambda s, sref: (0, 0)),
            pl.BlockSpec((H, C), lambda s, sref: (0, 0)),
            pl.BlockSpec((1, H), lambda s, sref: (0, 0)),
            pl.BlockSpec((1, C), lambda s, sref: (0, 0)),
        ],
        out_specs=pl.BlockSpec((BM, C),
                               lambda s, sref: (sref[s, _COL_OUTB], 0)),
        scratch_shapes=[
            pltpu.VMEM((NP, H), jnp.bfloat16),
            pltpu.VMEM((BM, H), jnp.float32),
            pltpu.VMEM((NP, C), jnp.bfloat16),
            pltpu.VMEM((NP, C), jnp.float32),
        ],
    )
    return pl.pallas_call(
        _gcn_kernel,
        grid_spec=grid_spec,
        out_shape=jax.ShapeDtypeStruct((N, C), jnp.float32),
    )(jnp.asarray(_SCHEDULE), adj, x, W1, W2,
      b1.reshape(1, H), b2.reshape(1, C))


# speedup vs baseline: 1.0061x; 1.0061x over previous
import functools

import numpy as np
import jax
import jax.numpy as jnp
from jax.experimental import pallas as pl
import jax.experimental.pallas.tpu as pltpu

N, F_IN, H, C = 10000, 128, 128, 64
K = 10
BM = 1024
NP = K * BM
EDGE = N - (K - 1) * BM

_COL_AR, _COL_AC, _COL_IB, _COL_L1, _COL_L1I, _COL_FIN, _COL_L2, _COL_L2I, \
    _COL_EMIT, _COL_OUTB = range(10)


def _build_schedule() -> np.ndarray:
    rows = []
    seen_l2 = set()

    def add(ar, ac, ib, l1, l1init, fin, l2, emit, outb):
        l2init = 1 if (l2 and ib not in seen_l2) else 0
        if l2:
            seen_l2.add(ib)
        rows.append([ar, ac, ib, l1, l1init, fin, l2, l2init, emit, outb])

    for i in range(K):
        for p in range(K):
            j = (i + 1 + p) % K
            fin = 1 if p == K - 1 else 0
            l2 = 1 if (j < i or fin) else 0
            add(i, j, i, 1, 1 if p == 0 else 0, fin, l2, 0, 0)
    for i in range(K - 1):
        for j in range(i + 1, K):
            add(i, j, i, 0, 0, 0, 1, 1 if j == K - 1 else 0, i)
    add(K - 2, K - 1, K - 1, 0, 0, 0, 0, 1, K - 1)
    return np.asarray(rows, dtype=np.int32)


_SCHEDULE = _build_schedule()
_STEPS = _SCHEDULE.shape[0]


def _gcn_kernel(sref, adj_ref, x_ref, w1_ref, w2_ref, b1_ref, b2_ref,
                out_ref, s1_ref, acc_ref, t_ref, oacc_ref):
    step = pl.program_id(0)
    ib = sref[step, _COL_IB]
    jb = sref[step, _COL_AC]

    @pl.when(step == 0)
    def _():
        s1_ref[pl.ds(0, N), :] = jnp.dot(
            x_ref[...], w1_ref[...], preferred_element_type=jnp.float32)
        s1_ref[pl.ds(N, NP - N), :] = jnp.zeros((NP - N, H), jnp.float32)

    def compute(adj):
        @pl.when(sref[step, _COL_L1] == 1)
        def _():
            c = jnp.dot(adj, s1_ref[pl.ds(jb * BM, BM), :],
                        preferred_element_type=jnp.float32)

            @pl.when(sref[step, _COL_L1I] == 1)
            def _():
                acc_ref[...] = c

            @pl.when(sref[step, _COL_L1I] == 0)
            def _():
                acc_ref[...] += c

        @pl.when(sref[step, _COL_FIN] == 1)
        def _():
            h1 = jnp.maximum(acc_ref[...] + b1_ref[...], 0.0)
            tv = jnp.dot(h1, w2_ref[...], preferred_element_type=jnp.float32)
            row_limit = jnp.where(ib == K - 1, EDGE, BM)
            row_ok = jax.lax.broadcasted_iota(jnp.int32, (BM, C), 0) < row_limit
            t_ref[pl.ds(ib * BM, BM), :] = jnp.where(row_ok, tv, 0.0)

        @pl.when(sref[step, _COL_L2] == 1)
        def _():
            c2 = jnp.dot(adj, t_ref[pl.ds(jb * BM, BM), :],
                         preferred_element_type=jnp.float32)

            @pl.when(sref[step, _COL_L2I] == 1)
            def _():
                oacc_ref[pl.ds(ib * BM, BM), :] = c2

            @pl.when(sref[step, _COL_L2I] == 0)
            def _():
                oacc_ref[pl.ds(ib * BM, BM), :] += c2

    @pl.when(jb == K - 1)
    def _():
        col_ok = jax.lax.broadcasted_iota(jnp.int32, (BM, BM), 1) < EDGE
        compute(jnp.where(col_ok, adj_ref[...], 0.0))

    @pl.when(jb != K - 1)
    def _():
        compute(adj_ref[...])

    @pl.when(sref[step, _COL_EMIT] == 1)
    def _():
        z = oacc_ref[pl.ds(ib * BM, BM), :] + b2_ref[...]
        zs = z - jnp.max(z, axis=1, keepdims=True)
        out_ref[...] = zs - jnp.log(
            jnp.sum(jnp.exp(zs), axis=1, keepdims=True))


@jax.jit
def kernel(x, adj, W1, b1, W2, b2):
    grid_spec = pltpu.PrefetchScalarGridSpec(
        num_scalar_prefetch=1,
        grid=(_STEPS,),
        in_specs=[
            pl.BlockSpec((BM, BM),
                         lambda s, sref: (sref[s, _COL_AR], sref[s, _COL_AC])),
            pl.BlockSpec((N, F_IN), lambda s, sref: (0, 0)),
            pl.BlockSpec((F_IN, H), lambda s, sref: (0, 0)),
            pl.BlockSpec((H, C), lambda s, sref: (0, 0)),
            pl.BlockSpec((1, H), lambda s, sref: (0, 0)),
            pl.BlockSpec((1, C), lambda s, sref: (0, 0)),
        ],
        out_specs=pl.BlockSpec((BM, C),
                               lambda s, sref: (sref[s, _COL_OUTB], 0)),
        scratch_shapes=[
            pltpu.VMEM((NP, H), jnp.float32),
            pltpu.VMEM((BM, H), jnp.float32),
            pltpu.VMEM((NP, C), jnp.float32),
            pltpu.VMEM((NP, C), jnp.float32),
        ],
    )
    return pl.pallas_call(
        _gcn_kernel,
        grid_spec=grid_spec,
        out_shape=jax.ShapeDtypeStruct((N, C), jnp.float32),
    )(jnp.asarray(_SCHEDULE), adj, x, W1, W2,
      b1.reshape(1, H), b2.reshape(1, C))


# speedup vs baseline: 1.1072x; 1.1005x over previous
import functools

import numpy as np
import jax
import jax.numpy as jnp
from jax.experimental import pallas as pl
import jax.experimental.pallas.tpu as pltpu

N, F_IN, H, C = 10000, 128, 128, 64
K = 10
BM = 1024
NP = K * BM
EDGE = N - (K - 1) * BM
R = H + C

_COL_AR, _COL_AC, _COL_IB, _COL_L1, _COL_L1I, _COL_FIN, _COL_L2, _COL_L2I, \
    _COL_EMIT, _COL_OUTB = range(10)


def _build_schedule() -> np.ndarray:
    rows = []
    seen_l2 = set()

    def add(ar, ac, ib, l1, l1init, fin, l2, emit, outb):
        l2init = 1 if (l2 and ib not in seen_l2) else 0
        if l2:
            seen_l2.add(ib)
        rows.append([ar, ac, ib, l1, l1init, fin, l2, l2init, emit, outb])

    for i in range(K):
        for p in range(K):
            j = (i + 1 + p) % K
            fin = 1 if p == K - 1 else 0
            l2 = 1 if (j < i or fin) else 0
            add(i, j, i, 1, 1 if p == 0 else 0, fin, l2, 0, 0)
    for i in range(K - 1):
        for j in range(i + 1, K):
            add(i, j, i, 0, 0, 0, 1, 1 if j == K - 1 else 0, i)
    add(K - 2, K - 1, K - 1, 0, 0, 0, 0, 1, K - 1)
    return np.asarray(rows, dtype=np.int32)


_SCHEDULE = _build_schedule()
_STEPS = _SCHEDULE.shape[0]


def _gcn_kernel(sref, adj_ref, x_ref, w1_ref, w2_ref, b1_ref, b2_ref,
                out_ref, rhs_ref, acc_ref, oacc_ref):
    step = pl.program_id(0)
    ib = sref[step, _COL_IB]
    jb = sref[step, _COL_AC]

    @pl.when(step == 0)
    def _():
        rhs_ref[pl.ds(0, N), pl.ds(0, H)] = jnp.dot(
            x_ref[...], w1_ref[...], preferred_element_type=jnp.float32)
        rhs_ref[pl.ds(N, NP - N), pl.ds(0, H)] = jnp.zeros(
            (NP - N, H), jnp.float32)

    def compute(adj):
        res = jnp.dot(adj, rhs_ref[pl.ds(jb * BM, BM), :],
                      preferred_element_type=jnp.float32)

        @pl.when(sref[step, _COL_L1] == 1)
        def _():
            c = res[:, 0:H]

            @pl.when(sref[step, _COL_L1I] == 1)
            def _():
                acc_ref[...] = c

            @pl.when(sref[step, _COL_L1I] == 0)
            def _():
                acc_ref[...] += c

        def oacc_update(c2):
            @pl.when(sref[step, _COL_L2I] == 1)
            def _():
                oacc_ref[pl.ds(ib * BM, BM), :] = c2

            @pl.when(sref[step, _COL_L2I] == 0)
            def _():
                oacc_ref[pl.ds(ib * BM, BM), :] += c2

        @pl.when(sref[step, _COL_FIN] == 1)
        def _():
            h1 = jnp.maximum(acc_ref[...] + b1_ref[...], 0.0)
            tv = jnp.dot(h1, w2_ref[...], preferred_element_type=jnp.float32)
            row_limit = jnp.where(ib == K - 1, EDGE, BM)
            row_ok = jax.lax.broadcasted_iota(jnp.int32, (BM, C), 0) < row_limit
            tm = jnp.where(row_ok, tv, 0.0)
            rhs_ref[pl.ds(ib * BM, BM), pl.ds(H, C)] = tm
            oacc_update(jnp.dot(adj, tm, preferred_element_type=jnp.float32))

        @pl.when((sref[step, _COL_L2] == 1) & (sref[step, _COL_FIN] == 0))
        def _():
            oacc_update(res[:, H:R])

    @pl.when(jb == K - 1)
    def _():
        col_ok = jax.lax.broadcasted_iota(jnp.int32, (BM, BM), 1) < EDGE
        compute(jnp.where(col_ok, adj_ref[...], 0.0))

    @pl.when(jb != K - 1)
    def _():
        compute(adj_ref[...])

    @pl.when(sref[step, _COL_EMIT] == 1)
    def _():
        z = oacc_ref[pl.ds(ib * BM, BM), :] + b2_ref[...]
        zs = z - jnp.max(z, axis=1, keepdims=True)
        out_ref[...] = zs - jnp.log(
            jnp.sum(jnp.exp(zs), axis=1, keepdims=True))


@jax.jit
def kernel(x, adj, W1, b1, W2, b2):
    grid_spec = pltpu.PrefetchScalarGridSpec(
        num_scalar_prefetch=1,
        grid=(_STEPS,),
        in_specs=[
            pl.BlockSpec((BM, BM),
                         lambda s, sref: (sref[s, _COL_AR], sref[s, _COL_AC])),
            pl.BlockSpec((N, F_IN), lambda s, sref: (0, 0)),
            pl.BlockSpec((F_IN, H), lambda s, sref: (0, 0)),
            pl.BlockSpec((H, C), lambda s, sref: (0, 0)),
            pl.BlockSpec((1, H), lambda s, sref: (0, 0)),
            pl.BlockSpec((1, C), lambda s, sref: (0, 0)),
        ],
        out_specs=pl.BlockSpec((BM, C),
                               lambda s, sref: (sref[s, _COL_OUTB], 0)),
        scratch_shapes=[
            pltpu.VMEM((NP, R), jnp.float32),
            pltpu.VMEM((BM, H), jnp.float32),
            pltpu.VMEM((NP, C), jnp.float32),
        ],
    )
    return pl.pallas_call(
        _gcn_kernel,
        grid_spec=grid_spec,
        out_shape=jax.ShapeDtypeStruct((N, C), jnp.float32),
    )(jnp.asarray(_SCHEDULE), adj, x, W1, W2,
      b1.reshape(1, H), b2.reshape(1, C))


# speedup vs baseline: 1.3526x; 1.2217x over previous
import functools

import numpy as np
import jax
import jax.numpy as jnp
from jax.experimental import pallas as pl
import jax.experimental.pallas.tpu as pltpu

N, F_IN, H, C = 10000, 128, 128, 64
KR, BMR = 10, 1024
KC, BN = 5, 2048
G = BN // BMR
NP = KR * BMR
EDGE_R = N - (KR - 1) * BMR
EDGE_C = N - (KC - 1) * BN
R = H + C

_COL_AR, _COL_AC, _COL_IB, _COL_L1, _COL_L1I, _COL_FIN, _COL_L2R, _COL_FL2, \
    _COL_L2I, _COL_EMIT, _COL_OUTB = range(11)


def _build_schedule() -> np.ndarray:
    rows = []
    seen_l2 = set()

    def add(ar, ac, ib, l1, l1init, fin, l2res, finl2, emit, outb):
        l2init = 1 if ((l2res or finl2) and ib not in seen_l2) else 0
        if l2res or finl2:
            seen_l2.add(ib)
        rows.append(
            [ar, ac, ib, l1, l1init, fin, l2res, finl2, l2init, emit, 0])

    def ready(c, i):
        return G * c + G - 1 < i

    def diag_ok(c, i):
        return c == i // G and i % G == G - 1

    for i in range(KR):
        cd = i // G
        for p in range(KC):
            c = (cd + 1 + p) % KC
            fin = 1 if p == KC - 1 else 0
            add(i, c, i, 1, 1 if p == 0 else 0, fin,
                1 if ready(c, i) else 0,
                1 if (fin and diag_ok(c, i)) else 0, 0, 0)
    for i in range(KR):
        cset = [c for c in range(KC) if not (ready(c, i) or diag_ok(c, i))]
        for idx, c in enumerate(cset):
            add(i, c, i, 0, 0, 0, 1, 0, 1 if idx == len(cset) - 1 else 0, i)
        if not cset:
            add(KR - 2, KC - 1, i, 0, 0, 0, 0, 0, 1, i)

    arr = np.asarray(rows, dtype=np.int32)
    outb = KR - 1
    for s in range(arr.shape[0] - 1, -1, -1):
        if arr[s, _COL_EMIT]:
            outb = arr[s, _COL_IB]
        arr[s, _COL_OUTB] = outb
    return arr


_SCHEDULE = _build_schedule()
_STEPS = _SCHEDULE.shape[0]


def _gcn_kernel(sref, adj_ref, x_ref, w1_ref, w2_ref, b1_ref, b2_ref,
                out_ref, rhs_ref, acc_ref, oacc_ref):
    step = pl.program_id(0)
    ib = sref[step, _COL_IB]
    jb = sref[step, _COL_AC]

    @pl.when(step == 0)
    def _():
        rhs_ref[pl.ds(0, N), pl.ds(0, H)] = jnp.dot(
            x_ref[...], w1_ref[...], preferred_element_type=jnp.float32)
        rhs_ref[pl.ds(N, NP - N), pl.ds(0, H)] = jnp.zeros(
            (NP - N, H), jnp.float32)

    def compute(adj):
        res = jnp.dot(adj, rhs_ref[pl.ds(jb * BN, BN), :],
                      preferred_element_type=jnp.float32)

        @pl.when(sref[step, _COL_L1] == 1)
        def _():
            c = res[:, 0:H]

            @pl.when(sref[step, _COL_L1I] == 1)
            def _():
                acc_ref[...] = c

            @pl.when(sref[step, _COL_L1I] == 0)
            def _():
                acc_ref[...] += c

        def oacc_update(c2):
            @pl.when(sref[step, _COL_L2I] == 1)
            def _():
                oacc_ref[pl.ds(ib * BMR, BMR), :] = c2

            @pl.when(sref[step, _COL_L2I] == 0)
            def _():
                oacc_ref[pl.ds(ib * BMR, BMR), :] += c2

        @pl.when(sref[step, _COL_FIN] == 1)
        def _():
            h1 = jnp.maximum(acc_ref[...] + b1_ref[...], 0.0)
            tv = jnp.dot(h1, w2_ref[...], preferred_element_type=jnp.float32)
            row_limit = jnp.where(ib == KR - 1, EDGE_R, BMR)
            row_ok = jax.lax.broadcasted_iota(
                jnp.int32, (BMR, C), 0) < row_limit
            rhs_ref[pl.ds(ib * BMR, BMR), pl.ds(H, C)] = jnp.where(
                row_ok, tv, 0.0)

            @pl.when(sref[step, _COL_FL2] == 1)
            def _():
                oacc_update(jnp.dot(
                    adj, rhs_ref[pl.ds(jb * BN, BN), pl.ds(H, C)],
                    preferred_element_type=jnp.float32))

        @pl.when(sref[step, _COL_L2R] == 1)
        def _():
            oacc_update(res[:, H:R])

    @pl.when(jb == KC - 1)
    def _():
        col_ok = jax.lax.broadcasted_iota(jnp.int32, (BMR, BN), 1) < EDGE_C
        compute(jnp.where(col_ok, adj_ref[...], 0.0))

    @pl.when(jb != KC - 1)
    def _():
        compute(adj_ref[...])

    @pl.when(sref[step, _COL_EMIT] == 1)
    def _():
        z = oacc_ref[pl.ds(ib * BMR, BMR), :] + b2_ref[...]
        zs = z - jnp.max(z, axis=1, keepdims=True)
        out_ref[...] = zs - jnp.log(
            jnp.sum(jnp.exp(zs), axis=1, keepdims=True))


@jax.jit
def kernel(x, adj, W1, b1, W2, b2):
    grid_spec = pltpu.PrefetchScalarGridSpec(
        num_scalar_prefetch=1,
        grid=(_STEPS,),
        in_specs=[
            pl.BlockSpec((BMR, BN),
                         lambda s, sref: (sref[s, _COL_AR], sref[s, _COL_AC])),
            pl.BlockSpec((N, F_IN), lambda s, sref: (0, 0)),
            pl.BlockSpec((F_IN, H), lambda s, sref: (0, 0)),
            pl.BlockSpec((H, C), lambda s, sref: (0, 0)),
            pl.BlockSpec((1, H), lambda s, sref: (0, 0)),
            pl.BlockSpec((1, C), lambda s, sref: (0, 0)),
        ],
        out_specs=pl.BlockSpec((BMR, C),
                               lambda s, sref: (sref[s, _COL_OUTB], 0)),
        scratch_shapes=[
            pltpu.VMEM((NP, R), jnp.float32),
            pltpu.VMEM((BMR, H), jnp.float32),
            pltpu.VMEM((NP, C), jnp.float32),
        ],
    )
    return pl.pallas_call(
        _gcn_kernel,
        grid_spec=grid_spec,
        out_shape=jax.ShapeDtypeStruct((N, C), jnp.float32),
    )(jnp.asarray(_SCHEDULE), adj, x, W1, W2,
      b1.reshape(1, H), b2.reshape(1, C))


# speedup vs baseline: 1.3988x; 1.0342x over previous
import functools

import numpy as np
import jax
import jax.numpy as jnp
from jax.experimental import pallas as pl
import jax.experimental.pallas.tpu as pltpu

N, F_IN, H, C = 10000, 128, 128, 64
KR, BMR = 5, 2048
KC, BN = 5, 2048
G = BN // BMR
NP = KR * BMR
EDGE_R = N - (KR - 1) * BMR
EDGE_C = N - (KC - 1) * BN
R = H + C
OA = R

_COL_AR, _COL_AC, _COL_IB, _COL_L1, _COL_L1I, _COL_FIN, _COL_L2R, _COL_FL2, \
    _COL_L2I, _COL_EMIT, _COL_OUTB = range(11)


def _build_schedule() -> np.ndarray:
    rows = []
    seen_l2 = set()

    def add(ar, ac, ib, l1, l1init, fin, l2res, finl2, emit, outb):
        l2init = 1 if ((l2res or finl2) and ib not in seen_l2) else 0
        if l2res or finl2:
            seen_l2.add(ib)
        rows.append(
            [ar, ac, ib, l1, l1init, fin, l2res, finl2, l2init, emit, 0])

    def ready(c, i):
        return G * c + G - 1 < i

    def diag_ok(c, i):
        return c == i // G and i % G == G - 1

    for i in range(KR):
        cd = i // G
        for p in range(KC):
            c = (cd + 1 + p) % KC
            fin = 1 if p == KC - 1 else 0
            add(i, c, i, 1, 1 if p == 0 else 0, fin,
                1 if ready(c, i) else 0,
                1 if (fin and diag_ok(c, i)) else 0, 0, 0)
    for i in range(KR):
        cset = [c for c in range(KC) if not (ready(c, i) or diag_ok(c, i))]
        for idx, c in enumerate(cset):
            add(i, c, i, 0, 0, 0, 1, 0, 1 if idx == len(cset) - 1 else 0, i)
        if not cset:
            add(KR - 2, KC - 1, i, 0, 0, 0, 0, 0, 1, i)

    arr = np.asarray(rows, dtype=np.int32)
    outb = KR - 1
    for s in range(arr.shape[0] - 1, -1, -1):
        if arr[s, _COL_EMIT]:
            outb = arr[s, _COL_IB]
        arr[s, _COL_OUTB] = outb
    return arr


_SCHEDULE = _build_schedule()
_STEPS = _SCHEDULE.shape[0]


def _gcn_kernel(sref, adj_ref, x_ref, w1_ref, w2_ref, b1_ref, b2_ref,
                out_ref, rhs_ref, acc_ref):
    step = pl.program_id(0)
    ib = sref[step, _COL_IB]
    jb = sref[step, _COL_AC]

    @pl.when(step == 0)
    def _():
        rhs_ref[pl.ds(0, N), pl.ds(0, H)] = jnp.dot(
            x_ref[...], w1_ref[...], preferred_element_type=jnp.float32)
        rhs_ref[pl.ds(N, NP - N), pl.ds(0, H)] = jnp.zeros(
            (NP - N, H), jnp.float32)

    def compute():
        res = jnp.dot(adj_ref[...], rhs_ref[pl.ds(jb * BN, BN), pl.ds(0, R)],
                      preferred_element_type=jnp.float32)

        @pl.when(sref[step, _COL_L1] == 1)
        def _():
            c = res[:, 0:H]

            @pl.when(sref[step, _COL_L1I] == 1)
            def _():
                acc_ref[...] = c

            @pl.when(sref[step, _COL_L1I] == 0)
            def _():
                acc_ref[...] += c

        def oacc_update(c2):
            @pl.when(sref[step, _COL_L2I] == 1)
            def _():
                rhs_ref[pl.ds(ib * BMR, BMR), pl.ds(OA, C)] = c2

            @pl.when(sref[step, _COL_L2I] == 0)
            def _():
                rhs_ref[pl.ds(ib * BMR, BMR), pl.ds(OA, C)] += c2

        @pl.when(sref[step, _COL_FIN] == 1)
        def _():
            h1 = jnp.maximum(acc_ref[...] + b1_ref[...], 0.0)
            tv = jnp.dot(h1, w2_ref[...], preferred_element_type=jnp.float32)
            row_limit = jnp.where(ib == KR - 1, EDGE_R, BMR)
            row_ok = jax.lax.broadcasted_iota(
                jnp.int32, (BMR, C), 0) < row_limit
            rhs_ref[pl.ds(ib * BMR, BMR), pl.ds(H, C)] = jnp.where(
                row_ok, tv, 0.0)

            @pl.when(sref[step, _COL_FL2] == 1)
            def _():
                oacc_update(jnp.dot(
                    adj_ref[...], rhs_ref[pl.ds(jb * BN, BN), pl.ds(H, C)],
                    preferred_element_type=jnp.float32))

        @pl.when(sref[step, _COL_L2R] == 1)
        def _():
            oacc_update(res[:, H:R])

    compute()

    @pl.when(sref[step, _COL_EMIT] == 1)
    def _():
        z = rhs_ref[pl.ds(ib * BMR, BMR), pl.ds(OA, C)] + b2_ref[...]
        zs = z - jnp.max(z, axis=1, keepdims=True)
        out_ref[...] = zs - jnp.log(
            jnp.sum(jnp.exp(zs), axis=1, keepdims=True))


@jax.jit
def kernel(x, adj, W1, b1, W2, b2):
    grid_spec = pltpu.PrefetchScalarGridSpec(
        num_scalar_prefetch=1,
        grid=(_STEPS,),
        in_specs=[
            pl.BlockSpec((BMR, BN),
                         lambda s, sref: (sref[s, _COL_AR], sref[s, _COL_AC]),
                         pipeline_mode=pl.Buffered(buffer_count=2)),
            pl.BlockSpec((N, F_IN), lambda s, sref: (0, 0)),
            pl.BlockSpec((F_IN, H), lambda s, sref: (0, 0)),
            pl.BlockSpec((H, C), lambda s, sref: (0, 0)),
            pl.BlockSpec((1, H), lambda s, sref: (0, 0)),
            pl.BlockSpec((1, C), lambda s, sref: (0, 0)),
        ],
        out_specs=pl.BlockSpec((BMR, C),
                               lambda s, sref: (sref[s, _COL_OUTB], 0)),
        scratch_shapes=[
            pltpu.VMEM((NP, OA + C), jnp.float32),
            pltpu.VMEM((BMR, H), jnp.float32),
        ],
    )
    return pl.pallas_call(
        _gcn_kernel,
        grid_spec=grid_spec,
        out_shape=jax.ShapeDtypeStruct((N, C), jnp.float32),
    )(jnp.asarray(_SCHEDULE), adj, x.astype(jnp.bfloat16),
      W1.astype(jnp.bfloat16), W2,
      b1.reshape(1, H), b2.reshape(1, C))
